# baseline (speedup 1.0000x reference)
import functools

import jax
import jax.numpy as jnp
from jax import lax
from jax.experimental import pallas as pl
from jax.experimental.pallas import tpu as pltpu
from jax.experimental.pallas import tpu_sc as plsc

_PROWS_BITS = 17
_PROWS = 1 << _PROWS_BITS


def _make_project_body(pack):
    def _project_body(*refs):
        tt_refs, wtt_ref, o_ref = refs[:pack], refs[pack], refs[pack + 1]
        ys = [
            jnp.dot(wtt_ref[...], tt_refs[e][...],
                    preferred_element_type=jnp.float32)
            for e in range(pack)
        ]
        y = jnp.concatenate(ys, axis=0)
        o_ref[...] = y.T

    return _project_body


def _project(tableT, WTT, block_cols=4096):
    D, V = tableT.shape
    pack = 128 // WTT.shape[0]
    n_i = _PROWS // block_cols
    max_blk = (V + block_cols - 1) // block_cols - 1

    t_specs = [
        pl.BlockSpec(
            (D, block_cols),
            lambda i, e=e, n=n_i, m=max_blk: (0, jnp.minimum(e * n + i, m)))
        for e in range(pack)
    ]
    return pl.pallas_call(
        _make_project_body(pack),
        grid=(n_i,),
        in_specs=t_specs + [
            pl.BlockSpec((WTT.shape[0], D), lambda i: (0, 0)),
        ],
        out_specs=pl.BlockSpec((block_cols, 128), lambda i: (i, 0)),
        out_shape=jax.ShapeDtypeStruct((_PROWS, 128), jnp.float32),
    )(tableT, *([tableT] * (pack - 1)), WTT)


_K = 128
_L = 16


def _make_gather(B, C, num_cores=2, num_subcores=16):
    NW = num_cores * num_subcores
    pack = 128 // C
    b_per_w = B // NW
    chunks = b_per_w // _K
    out_rows = _K // pack
    mesh = plsc.VectorSubcoreMesh(
        core_axis_name="c", subcore_axis_name="s",
        num_cores=num_cores, num_subcores=num_subcores)

    @functools.partial(
        pl.kernel,
        out_type=jax.ShapeDtypeStruct((B // pack, 128), jnp.float32),
        mesh=mesh,
        scratch_types=[
            pltpu.VMEM((chunks, _K), jnp.int32),
            pltpu.VMEM((4, _K), jnp.int32),
            pltpu.VMEM((4, _K, 128), jnp.float32),
            pltpu.VMEM((out_rows, 128), jnp.float32),
            pltpu.SemaphoreType.DMA,
            pltpu.SemaphoreType.DMA,
            pltpu.SemaphoreType.DMA,
            pltpu.SemaphoreType.DMA,
        ],
    )
    def gather(p_hbm, idx_hbm, out_hbm, idx_v, hi_v, rows_v, out_v,
               sem0, sem1, sem2, sem3):
        wid = lax.axis_index("s") * num_cores + lax.axis_index("c")
        out_base = wid * chunks * out_rows
        pltpu.sync_copy(idx_hbm.at[pl.ds(wid * chunks, chunks)], idx_v)

        def fire(j, s, sem):
            for g in range(_K // _L):
                idx16 = idx_v[j, pl.ds(g * _L, _L)]
                hi_v[s, pl.ds(g * _L, _L)] = idx16 & (_PROWS - 1)
            pltpu.async_copy(p_hbm.at[hi_v.at[s]], rows_v.at[s], sem)

        def process(j, s, sem):
            pltpu.make_async_copy(
                p_hbm.at[hi_v.at[s]], rows_v.at[s], sem).wait()

            def extract(g, _):
                idx16 = idx_v[j, pl.ds(g * _L, _L)]
                for u in range(_L):
                    kk = g * _L + u
                    lo = lax.shift_right_logical(idx16[u], _PROWS_BITS) * C
                    x = rows_v[s, kk, pl.ds(lo, C)]
                    out_v[kk // pack, pl.ds((kk % pack) * C, C)] = x
                return 0

            lax.fori_loop(0, _K // _L, extract, 0, unroll=False)
            pltpu.sync_copy(
                out_v, out_hbm.at[pl.ds(out_base + j * out_rows, out_rows)])

        sems = (sem0, sem1, sem2, sem3)
        fire(0, 0, sem0)
        fire(1, 1, sem1)

        def body(t, _):
            j0 = t * 4
            for u in range(4):
                nxt = j0 + u + 2

                @pl.when(nxt < chunks)
                def _prefetch(nxt=nxt, u=u):
                    fire(nxt, (u + 2) % 4, sems[(u + 2) % 4])

                process(j0 + u, u, sems[u])
            return 0

        lax.fori_loop(0, chunks // 4, body, 0, unroll=False)

    return gather


def kernel(indices, table, W, b):
    Bb, S = indices.shape
    V, D = table.shape
    C = W.shape[0]
    P = _project(table.T, W)
    idx2 = indices.reshape(-1, _K)
    out = _make_gather(Bb * S, C)(P, idx2)
    return out.reshape(Bb, S, C) + b[None, None, :]


# speedup vs baseline: 1.3582x; 1.3582x over previous
import functools

import jax
import jax.numpy as jnp
from jax import lax
from jax.experimental import pallas as pl
from jax.experimental.pallas import tpu as pltpu
from jax.experimental.pallas import tpu_sc as plsc

_PROWS_BITS = 17
_PROWS = 1 << _PROWS_BITS


def _make_project_body(pack):
    def _project_body(*refs):
        tt_refs, wtt_ref, b_ref, o_ref = (
            refs[:pack], refs[pack], refs[pack + 1], refs[pack + 2])
        ys = [
            jnp.dot(wtt_ref[...], tt_refs[e][...],
                    preferred_element_type=jnp.float32)
            for e in range(pack)
        ]
        y = jnp.concatenate(ys, axis=0)
        o_ref[...] = y.T + b_ref[...]

    return _project_body


def _project(tableT, WTT, b128, block_cols=4096):
    D, V = tableT.shape
    pack = 128 // WTT.shape[0]
    n_i = _PROWS // block_cols
    max_blk = (V + block_cols - 1) // block_cols - 1

    t_specs = [
        pl.BlockSpec(
            (D, block_cols),
            lambda i, e=e, n=n_i, m=max_blk: (0, jnp.minimum(e * n + i, m)))
        for e in range(pack)
    ]
    return pl.pallas_call(
        _make_project_body(pack),
        grid=(n_i,),
        in_specs=t_specs + [
            pl.BlockSpec((WTT.shape[0], D), lambda i: (0, 0)),
            pl.BlockSpec((1, 128), lambda i: (0, 0)),
        ],
        out_specs=pl.BlockSpec((block_cols, 128), lambda i: (i, 0)),
        out_shape=jax.ShapeDtypeStruct((_PROWS, 128), jnp.float32),
    )(tableT, *([tableT] * (pack - 1)), WTT, b128)


_K = 128
_L = 16


def _make_gather(B, C, num_cores=2, num_subcores=16):
    NW = num_cores * num_subcores
    pack = 128 // C
    b_per_w = B // NW
    chunks = b_per_w // _K
    out_rows = _K // pack
    mesh = plsc.VectorSubcoreMesh(
        core_axis_name="c", subcore_axis_name="s",
        num_cores=num_cores, num_subcores=num_subcores)

    @functools.partial(
        pl.kernel,
        out_type=jax.ShapeDtypeStruct((B // pack, 128), jnp.float32),
        mesh=mesh,
        cost_estimate=pl.CostEstimate(
            flops=0, bytes_accessed=500_000_000, transcendentals=0),
        scratch_types=[
            pltpu.VMEM((chunks, _K), jnp.int32),
            pltpu.VMEM((4, _K), jnp.int32),
            pltpu.VMEM((4, _K, 128), jnp.float32),
            pltpu.VMEM((out_rows, 128), jnp.float32),
            pltpu.SemaphoreType.DMA,
            pltpu.SemaphoreType.DMA,
            pltpu.SemaphoreType.DMA,
            pltpu.SemaphoreType.DMA,
        ],
    )
    def gather(p_hbm, idx_hbm, out_hbm, idx_v, hi_v, rows_v, out_v,
               sem0, sem1, sem2, sem3):
        wid = lax.axis_index("s") * num_cores + lax.axis_index("c")
        out_base = wid * chunks * out_rows
        pltpu.sync_copy(idx_hbm.at[pl.ds(wid * chunks, chunks)], idx_v)

        def fire(j, s, sem):
            for g in range(_K // _L):
                idx16 = idx_v[j, pl.ds(g * _L, _L)]
                hi_v[s, pl.ds(g * _L, _L)] = idx16 & (_PROWS - 1)
            pltpu.async_copy(p_hbm.at[hi_v.at[s]], rows_v.at[s], sem)

        def process(j, s, sem):
            pltpu.make_async_copy(
                p_hbm.at[hi_v.at[s]], rows_v.at[s], sem).wait()

            def extract(g, _):
                idx16 = idx_v[j, pl.ds(g * _L, _L)]
                for u in range(_L):
                    kk = g * _L + u
                    lo = lax.shift_right_logical(idx16[u], _PROWS_BITS) * C
                    x = rows_v[s, kk, pl.ds(lo, C)]
                    out_v[kk // pack, pl.ds((kk % pack) * C, C)] = x
                return 0

            lax.fori_loop(0, _K // _L, extract, 0, unroll=False)
            pltpu.sync_copy(
                out_v, out_hbm.at[pl.ds(out_base + j * out_rows, out_rows)])

        sems = (sem0, sem1, sem2, sem3)
        fire(0, 0, sem0)
        fire(1, 1, sem1)

        def body(t, _):
            j0 = t * 4
            for u in range(4):
                nxt = j0 + u + 2

                @pl.when(nxt < chunks)
                def _prefetch(nxt=nxt, u=u):
                    fire(nxt, (u + 2) % 4, sems[(u + 2) % 4])

                process(j0 + u, u, sems[u])
            return 0

        lax.fori_loop(0, chunks // 4, body, 0, unroll=False)

    return gather


def kernel(indices, table, W, b):
    Bb, S = indices.shape
    V, D = table.shape
    C = W.shape[0]
    P = _project(table.T, W, jnp.tile(b, 128 // C).reshape(1, 128))
    idx2 = indices.reshape(-1, _K)
    out = _make_gather(Bb * S, C)(P, idx2)
    return out.reshape(Bb, S, C)


# speedup vs baseline: 2.6108x; 1.9222x over previous
import functools

import jax
import jax.numpy as jnp
from jax import lax
from jax.experimental import pallas as pl
from jax.experimental.pallas import tpu as pltpu
from jax.experimental.pallas import tpu_sc as plsc

_PROWS_BITS = 17
_PROWS = 1 << _PROWS_BITS


def _make_project_body(pack):
    def _project_body(*refs):
        tt_refs, wtt_ref, b_ref, o_ref = (
            refs[:pack], refs[pack], refs[pack + 1], refs[pack + 2])
        ys = [
            jnp.dot(wtt_ref[...], tt_refs[e][...],
                    preferred_element_type=jnp.float32)
            for e in range(pack)
        ]
        y = jnp.concatenate(ys, axis=0)
        o_ref[...] = y.T + b_ref[...]

    return _project_body


def _project(tableT, WTT, b128, block_cols=4096):
    D, V = tableT.shape
    pack = 128 // WTT.shape[0]
    n_i = _PROWS // block_cols
    max_blk = (V + block_cols - 1) // block_cols - 1

    t_specs = [
        pl.BlockSpec(
            (D, block_cols),
            lambda i, e=e, n=n_i, m=max_blk: (0, jnp.minimum(e * n + i, m)))
        for e in range(pack)
    ]
    return pl.pallas_call(
        _make_project_body(pack),
        grid=(n_i,),
        in_specs=t_specs + [
            pl.BlockSpec((WTT.shape[0], D), lambda i: (0, 0)),
            pl.BlockSpec((1, 128), lambda i: (0, 0)),
        ],
        out_specs=pl.BlockSpec((block_cols, 128), lambda i: (i, 0)),
        out_shape=jax.ShapeDtypeStruct((_PROWS, 128), jnp.float32),
    )(tableT, *([tableT] * (pack - 1)), WTT, b128)


def _xpose_body(x_ref, o_ref):
    o_ref[...] = x_ref[...].T


def _xpose(x, block_rows=512):
    R, Cc = x.shape
    return pl.pallas_call(
        _xpose_body,
        grid=(R // block_rows,),
        in_specs=[pl.BlockSpec((block_rows, Cc), lambda i: (i, 0))],
        out_specs=pl.BlockSpec((Cc, block_rows), lambda i: (0, i)),
        out_shape=jax.ShapeDtypeStruct((Cc, R), x.dtype),
    )(x)


_K = 128
_L = 16


def _make_gather(B, C, num_cores=2, num_subcores=16):
    NW = num_cores * num_subcores
    pack = 128 // C
    b_per_w = B // NW
    chunks = b_per_w // _K
    out_rows = _K // pack
    mesh = plsc.VectorSubcoreMesh(
        core_axis_name="c", subcore_axis_name="s",
        num_cores=num_cores, num_subcores=num_subcores)

    @functools.partial(
        pl.kernel,
        out_type=jax.ShapeDtypeStruct((B // pack, 128), jnp.float32),
        mesh=mesh,
        cost_estimate=pl.CostEstimate(
            flops=0, bytes_accessed=500_000_000, transcendentals=0),
        scratch_types=[
            pltpu.VMEM((chunks, _K), jnp.int32),
            pltpu.VMEM((4, _K), jnp.int32),
            pltpu.VMEM((4, _K, 128), jnp.float32),
            pltpu.VMEM((out_rows, 128), jnp.float32),
            pltpu.SemaphoreType.DMA,
            pltpu.SemaphoreType.DMA,
            pltpu.SemaphoreType.DMA,
            pltpu.SemaphoreType.DMA,
        ],
    )
    def gather(p_hbm, idx_hbm, out_hbm, idx_v, hi_v, rows_v, out_v,
               sem0, sem1, sem2, sem3):
        wid = lax.axis_index("s") * num_cores + lax.axis_index("c")
        out_base = wid * chunks * out_rows
        pltpu.sync_copy(idx_hbm.at[pl.ds(wid * chunks, chunks)], idx_v)

        def fire(j, s, sem):
            for g in range(_K // _L):
                idx16 = idx_v[j, pl.ds(g * _L, _L)]
                hi_v[s, pl.ds(g * _L, _L)] = idx16 & (_PROWS - 1)
            pltpu.async_copy(p_hbm.at[hi_v.at[s]], rows_v.at[s], sem)

        def process(j, s, sem):
            pltpu.make_async_copy(
                p_hbm.at[hi_v.at[s]], rows_v.at[s], sem).wait()

            def extract(g, _):
                idx16 = idx_v[j, pl.ds(g * _L, _L)]
                for u in range(_L):
                    kk = g * _L + u
                    lo = lax.shift_right_logical(idx16[u], _PROWS_BITS) * C
                    x = rows_v[s, kk, pl.ds(lo, C)]
                    out_v[kk // pack, pl.ds((kk % pack) * C, C)] = x
                return 0

            lax.fori_loop(0, _K // _L, extract, 0, unroll=False)
            pltpu.sync_copy(
                out_v, out_hbm.at[pl.ds(out_base + j * out_rows, out_rows)])

        sems = (sem0, sem1, sem2, sem3)
        fire(0, 0, sem0)
        fire(1, 1, sem1)

        def body(t, _):
            j0 = t * 4
            for u in range(4):
                nxt = j0 + u + 2

                @pl.when(nxt < chunks)
                def _prefetch(nxt=nxt, u=u):
                    fire(nxt, (u + 2) % 4, sems[(u + 2) % 4])

                process(j0 + u, u, sems[u])
            return 0

        lax.fori_loop(0, chunks // 4, body, 0, unroll=False)

    return gather


def kernel(indices, table, W, b):
    Bb, S = indices.shape
    V, D = table.shape
    C = W.shape[0]
    P = _project(table.T, W, jnp.tile(b, 128 // C).reshape(1, 128))
    idx2 = indices.reshape(-1, _K)
    out = _make_gather(Bb * S, C)(P, idx2)
    outT = _xpose(out.reshape(Bb, S * C))
    return outT.reshape(S, C, Bb).transpose(2, 0, 1)


# speedup vs baseline: 2.6889x; 1.0299x over previous
import functools

import jax
import jax.numpy as jnp
from jax import lax
from jax.experimental import pallas as pl
from jax.experimental.pallas import tpu as pltpu
from jax.experimental.pallas import tpu_sc as plsc

_PROWS_BITS = 17
_PROWS = 1 << _PROWS_BITS


def _make_project_body(pack):
    def _project_body(*refs):
        tt_refs, wtt_ref, b_ref, o_ref = (
            refs[:pack], refs[pack], refs[pack + 1], refs[pack + 2])
        ys = [
            jnp.dot(wtt_ref[...], tt_refs[e][...],
                    preferred_element_type=jnp.float32)
            for e in range(pack)
        ]
        y = jnp.concatenate(ys, axis=0)
        o_ref[...] = y.T + b_ref[...]

    return _project_body


def _project(tableT, WTT, b128, block_cols=4096):
    D, V = tableT.shape
    pack = 128 // WTT.shape[0]
    n_i = _PROWS // block_cols
    max_blk = (V + block_cols - 1) // block_cols - 1

    t_specs = [
        pl.BlockSpec(
            (D, block_cols),
            lambda i, e=e, n=n_i, m=max_blk: (0, jnp.minimum(e * n + i, m)))
        for e in range(pack)
    ]
    return pl.pallas_call(
        _make_project_body(pack),
        grid=(n_i,),
        in_specs=t_specs + [
            pl.BlockSpec((WTT.shape[0], D), lambda i: (0, 0)),
            pl.BlockSpec((1, 128), lambda i: (0, 0)),
        ],
        out_specs=pl.BlockSpec((block_cols, 128), lambda i: (i, 0)),
        out_shape=jax.ShapeDtypeStruct((_PROWS, 128), jnp.float32),
    )(tableT, *([tableT] * (pack - 1)), WTT, b128)


def _xpose_body(x_ref, o_ref):
    o_ref[...] = x_ref[...].T


def _xpose(x, block_rows=512):
    R, Cc = x.shape
    return pl.pallas_call(
        _xpose_body,
        grid=(R // block_rows,),
        in_specs=[pl.BlockSpec((block_rows, Cc), lambda i: (i, 0))],
        out_specs=pl.BlockSpec((Cc, block_rows), lambda i: (0, i)),
        out_shape=jax.ShapeDtypeStruct((Cc, R), x.dtype),
    )(x)


_K = 128
_L = 16


def _make_gather(B, C, num_cores=2, num_subcores=16):
    NW = num_cores * num_subcores
    pack = 128 // C
    b_per_w = B // NW
    chunks = b_per_w // _K
    out_rows = _K // pack
    mesh = plsc.VectorSubcoreMesh(
        core_axis_name="c", subcore_axis_name="s",
        num_cores=num_cores, num_subcores=num_subcores)

    @functools.partial(
        pl.kernel,
        out_type=jax.ShapeDtypeStruct((B // pack, 128), jnp.float32),
        mesh=mesh,
        cost_estimate=pl.CostEstimate(
            flops=0, bytes_accessed=500_000_000, transcendentals=0),
        scratch_types=[
            pltpu.VMEM((chunks, _K), jnp.int32),
            pltpu.VMEM((4, _K), jnp.int32),
            pltpu.VMEM((4, _K, 128), jnp.float32),
            pltpu.VMEM((out_rows, 128), jnp.float32),
            pltpu.SemaphoreType.DMA,
            pltpu.SemaphoreType.DMA,
            pltpu.SemaphoreType.DMA,
            pltpu.SemaphoreType.DMA,
        ],
    )
    def gather(p_hbm, idx_hbm, out_hbm, idx_v, hi_v, rows_v, out_v,
               sem0, sem1, sem2, sem3):
        wid = lax.axis_index("s") * num_cores + lax.axis_index("c")
        out_base = wid * chunks * out_rows
        pltpu.sync_copy(idx_hbm.at[pl.ds(wid * chunks, chunks)], idx_v)

        def fire(j, s, sem):
            for g in range(_K // _L):
                idx16 = idx_v[j, pl.ds(g * _L, _L)]
                hi_v[s, pl.ds(g * _L, _L)] = idx16 & (_PROWS - 1)
            pltpu.async_copy(p_hbm.at[hi_v.at[s]], rows_v.at[s], sem)

        def process(j, s, sem):
            pltpu.make_async_copy(
                p_hbm.at[hi_v.at[s]], rows_v.at[s], sem).wait()

            def extract(g, _):
                idx16 = idx_v[j, pl.ds(g * _L, _L)]
                for u in range(_L):
                    kk = g * _L + u
                    lo = lax.shift_right_logical(idx16[u], _PROWS_BITS) * C
                    x = rows_v[s, kk, pl.ds(lo, C)]
                    out_v[kk // pack, pl.ds((kk % pack) * C, C)] = x
                return 0

            lax.fori_loop(0, _K // _L, extract, 0, unroll=False)
            pltpu.sync_copy(
                out_v, out_hbm.at[pl.ds(out_base + j * out_rows, out_rows)])

        sems = (sem0, sem1, sem2, sem3)
        fire(0, 0, sem0)
        fire(1, 1, sem1)
        fire(2, 2, sem2)

        def body(t, _):
            j0 = t * 4
            for u in range(4):
                nxt = j0 + u + 3

                @pl.when(nxt < chunks)
                def _prefetch(nxt=nxt, u=u):
                    fire(nxt, (u + 3) % 4, sems[(u + 3) % 4])

                process(j0 + u, u, sems[u])
            return 0

        lax.fori_loop(0, chunks // 4, body, 0, unroll=False)

    return gather


def kernel(indices, table, W, b):
    Bb, S = indices.shape
    V, D = table.shape
    C = W.shape[0]
    P = _project(table.T, W, jnp.tile(b, 128 // C).reshape(1, 128))
    idx2 = indices.reshape(-1, _K)
    out = _make_gather(Bb * S, C)(P, idx2)
    outT = _xpose(out.reshape(Bb, S * C))
    return outT.reshape(S, C, Bb).transpose(2, 0, 1)
